# baseline (speedup 1.0000x reference)
import functools

import jax
import jax.numpy as jnp
from jax import lax
from jax.experimental import pallas as pl
from jax.experimental.pallas import tpu as pltpu
from jax.experimental.pallas import tpu_sc as plsc


_BR = 256
_RC = 8
_CC = 512


def _tc_body(x_ref, o_ref):
    @pl.when(pl.program_id(0) == 0)
    def _init():
        o_ref[...] = jnp.zeros_like(o_ref)

    ncc = x_ref.shape[1] // _CC

    def row_step(i, acc):
        for c in range(ncc):
            t = x_ref[pl.ds(i * _RC, _RC), pl.ds(c * _CC, _CC)]
            acc = acc + (jnp.maximum(t, 0.0) + jnp.log1p(jnp.exp(-jnp.abs(t))))
        return acc

    acc = lax.fori_loop(
        0, x_ref.shape[0] // _RC, row_step,
        jnp.zeros((_RC, _CC), jnp.float32))
    o_ref[...] += jnp.sum(acc)


def _tc_softplus_sum(x):
    N, T = x.shape
    out = pl.pallas_call(
        _tc_body,
        grid=(N // _BR,),
        in_specs=[pl.BlockSpec((_BR, T), lambda i: (i, 0))],
        out_specs=pl.BlockSpec((1, 1), lambda i: (0, 0)),
        out_shape=jax.ShapeDtypeStruct((1, 1), jnp.float32),
    )(x)
    return out[0, 0]


_NW = 32
_T = 4096
_LANES = 16


_CHROWS = 2


def _sc_process_chunk(buf, acc):
    lanes = lax.broadcasted_iota(jnp.int32, (_LANES,), 0)
    neg_inf = jnp.full((_LANES,), -jnp.inf, jnp.float32)
    nsl = _CHROWS * 4
    m1 = [neg_inf] * nsl
    m2 = [neg_inf] * nsl
    for j in range(64):
        idxj = lanes * 64 + ((lanes + j) & 63)
        for k in range(nsl):
            sl = buf.at[pl.ds(k * 1024, 1024)]
            v = plsc.load_gather(sl, [idxj])
            t = jnp.minimum(m1[k], v)
            m1[k] = jnp.maximum(m1[k], v)
            m2[k] = jnp.maximum(m2[k], t)
    for k in range(nsl):
        acc = acc + jnp.where(m1[k] > 0.0, m1[k] + m2[k], 0.0)
    return acc


def _sc_kernel_body(x_hbm, out_hbm, buf0, buf1, accv, sem0, sem1):
    wid = lax.axis_index("s") * 2 + lax.axis_index("c")
    nrows = (x_hbm.shape[0] // _T) // _NW
    row0 = wid * nrows
    nch = nrows // _CHROWS
    T = _T

    def chunk_copy(r, buf, sem):
        return [pltpu.make_async_copy(
            x_hbm.at[pl.ds(r * T, _CHROWS * T)], buf, sem)]

    def start(copies):
        for c in copies:
            c.start()

    def drain(copies):
        for c in copies:
            c.wait()

    start(chunk_copy(row0, buf0, sem0))

    def step(i, acc):
        r = row0 + 2 * i * _CHROWS
        start(chunk_copy(r + _CHROWS, buf1, sem1))
        drain(chunk_copy(r, buf0, sem0))
        acc = _sc_process_chunk(buf0, acc)

        @pl.when(2 * i + 2 < nch)
        def _():
            start(chunk_copy(r + 2 * _CHROWS, buf0, sem0))

        drain(chunk_copy(r + _CHROWS, buf1, sem1))
        acc = _sc_process_chunk(buf1, acc)
        return acc

    acc = lax.fori_loop(0, nch // 2, step,
                        jnp.zeros((_LANES,), jnp.float32))
    accv[...] = acc
    pltpu.sync_copy(accv, out_hbm.at[wid])


def _sc_top2_partials(x):
    mesh = plsc.VectorSubcoreMesh(core_axis_name="c", subcore_axis_name="s")
    x = x.reshape(-1)
    T = _T
    run = pl.kernel(
        _sc_kernel_body,
        out_type=jax.ShapeDtypeStruct((_NW, _LANES), jnp.float32),
        mesh=mesh,
        scratch_types=[
            pltpu.VMEM((_CHROWS * T,), jnp.float32),
            pltpu.VMEM((_CHROWS * T,), jnp.float32),
            pltpu.VMEM((_LANES,), jnp.float32),
            pltpu.SemaphoreType.DMA,
            pltpu.SemaphoreType.DMA,
        ],
        compiler_params=pltpu.CompilerParams(needs_layout_passes=False),
    )
    return run(x)


def kernel(pred_logits):
    sp_sum = _tc_softplus_sum(pred_logits)
    parts = _sc_top2_partials(pred_logits)
    return sp_sum - jnp.sum(parts)


# speedup vs baseline: 1.6961x; 1.6961x over previous
import functools

import jax
import jax.numpy as jnp
from jax import lax
from jax.experimental import pallas as pl
from jax.experimental.pallas import tpu as pltpu
from jax.experimental.pallas import tpu_sc as plsc


_BR = 256
_RC = 8
_CC = 512


def _tc_body(x_ref, o_ref):
    @pl.when(pl.program_id(0) == 0)
    def _init():
        o_ref[...] = jnp.zeros_like(o_ref)

    ncc = x_ref.shape[1] // _CC

    def row_step(i, acc):
        for c in range(ncc):
            t = x_ref[pl.ds(i * _RC, _RC), pl.ds(c * _CC, _CC)]
            acc = acc + (jnp.maximum(t, 0.0) + jnp.log1p(jnp.exp(-jnp.abs(t))))
        return acc

    acc = lax.fori_loop(
        0, x_ref.shape[0] // _RC, row_step,
        jnp.zeros((_RC, _CC), jnp.float32))
    o_ref[...] += jnp.sum(acc)


def _tc_softplus_sum(x):
    N, T = x.shape
    out = pl.pallas_call(
        _tc_body,
        grid=(N // _BR,),
        in_specs=[pl.BlockSpec((_BR, T), lambda i: (i, 0))],
        out_specs=pl.BlockSpec((1, 1), lambda i: (0, 0)),
        out_shape=jax.ShapeDtypeStruct((1, 1), jnp.float32),
    )(x)
    return out[0, 0]


_NW = 32
_LANES = 16


_CHROWS = 2


def _sc_process_chunk(buf, acc):
    lanes = lax.broadcasted_iota(jnp.int32, (_LANES,), 0)
    neg_inf = jnp.full((_LANES,), -jnp.inf, jnp.float32)
    nsl = _CHROWS * 4
    m1 = [neg_inf] * nsl
    m2 = [neg_inf] * nsl
    for j in range(64):
        idxj = lanes * 64 + ((lanes + j) & 63)
        for k in range(nsl):
            sl = buf.at[pl.ds(k * 1024, 1024)]
            v = plsc.load_gather(sl, [idxj])
            t = jnp.minimum(m1[k], v)
            m1[k] = jnp.maximum(m1[k], v)
            m2[k] = jnp.maximum(m2[k], t)
    for k in range(nsl):
        acc = acc + jnp.where(m1[k] > 0.0, m1[k] + m2[k], 0.0)
    return acc


def _sc_kernel_body(x_hbm, out_hbm, buf0, buf1, accv, sem0, sem1):
    wid = lax.axis_index("s") * 2 + lax.axis_index("c")
    nrows = x_hbm.shape[0] // _NW
    row0 = wid * nrows
    nch = nrows // _CHROWS
    T = x_hbm.shape[1]

    def chunk_copy(r, buf, sem):
        copies = [
            pltpu.make_async_copy(
                x_hbm.at[r + q], buf.at[pl.ds(q * T, T)], sem)
            for q in range(_CHROWS)
        ]
        return copies

    def start(copies):
        for c in copies:
            c.start()

    def drain(copies):
        for c in copies:
            c.wait()

    start(chunk_copy(row0, buf0, sem0))

    def step(i, acc):
        r = row0 + 2 * i * _CHROWS
        start(chunk_copy(r + _CHROWS, buf1, sem1))
        drain(chunk_copy(r, buf0, sem0))
        acc = _sc_process_chunk(buf0, acc)

        @pl.when(2 * i + 2 < nch)
        def _():
            start(chunk_copy(r + 2 * _CHROWS, buf0, sem0))

        drain(chunk_copy(r + _CHROWS, buf1, sem1))
        acc = _sc_process_chunk(buf1, acc)
        return acc

    acc = lax.fori_loop(0, nch // 2, step,
                        jnp.zeros((_LANES,), jnp.float32))
    accv[...] = acc
    pltpu.sync_copy(accv, out_hbm.at[wid])


def _sc_top2_partials(x):
    mesh = plsc.VectorSubcoreMesh(core_axis_name="c", subcore_axis_name="s")
    T = x.shape[1]
    run = pl.kernel(
        _sc_kernel_body,
        out_type=jax.ShapeDtypeStruct((_NW, _LANES), jnp.float32),
        mesh=mesh,
        scratch_types=[
            pltpu.VMEM((_CHROWS * T,), jnp.float32),
            pltpu.VMEM((_CHROWS * T,), jnp.float32),
            pltpu.VMEM((_LANES,), jnp.float32),
            pltpu.SemaphoreType.DMA,
            pltpu.SemaphoreType.DMA,
        ],
        compiler_params=pltpu.CompilerParams(needs_layout_passes=False),
    )
    return run(x)


def kernel(pred_logits):
    parts = _sc_top2_partials(pred_logits)
    sp_sum = _tc_softplus_sum(pred_logits)
    return sp_sum - jnp.sum(parts)


# speedup vs baseline: 1.6998x; 1.0022x over previous
import jax
import jax.numpy as jnp
from jax import lax
from jax.experimental import pallas as pl
from jax.experimental.pallas import tpu as pltpu
from jax.experimental.pallas import tpu_sc as plsc


_BR = 256
_RC = 8
_CC = 512


def _tc_body(x_ref, o_ref):
    @pl.when(pl.program_id(0) == 0)
    def _init():
        o_ref[...] = jnp.zeros_like(o_ref)

    ncc = x_ref.shape[1] // _CC

    def row_step(i, acc):
        for c in range(ncc):
            t = x_ref[pl.ds(i * _RC, _RC), pl.ds(c * _CC, _CC)]
            acc = acc + (jnp.maximum(t, 0.0) + jnp.log1p(jnp.exp(-jnp.abs(t))))
        return acc

    acc = lax.fori_loop(
        0, x_ref.shape[0] // _RC, row_step,
        jnp.zeros((_RC, _CC), jnp.float32))
    o_ref[...] += jnp.sum(acc)


def _tc_softplus_sum(x):
    N, T = x.shape
    out = pl.pallas_call(
        _tc_body,
        grid=(N // _BR,),
        in_specs=[pl.BlockSpec((_BR, T), lambda i: (i, 0))],
        out_specs=pl.BlockSpec((1, 1), lambda i: (0, 0)),
        out_shape=jax.ShapeDtypeStruct((1, 1), jnp.float32),
    )(x)
    return out[0, 0]


_NW = 32
_LANES = 16


_CHROWS = 2


def _sc_process_chunk(buf, acc):
    lanes = lax.broadcasted_iota(jnp.int32, (_LANES,), 0)
    neg_inf = jnp.full((_LANES,), -jnp.inf, jnp.float32)
    nsl = _CHROWS * 4
    m1 = [neg_inf] * nsl
    m2 = [neg_inf] * nsl
    for j in range(64):
        idxj = lanes * 64 + ((lanes + j) & 63)
        for k in range(nsl):
            sl = buf.at[pl.ds(k * 1024, 1024)]
            v = plsc.load_gather(sl, [idxj])
            t = jnp.minimum(m1[k], v)
            m1[k] = jnp.maximum(m1[k], v)
            m2[k] = jnp.maximum(m2[k], t)
    for k in range(nsl):
        acc = acc + jnp.where(m1[k] > 0.0, m1[k] + m2[k], 0.0)
    return acc


def _sc_kernel_body(x_hbm, out_hbm, buf0, buf1, accv, sem0, sem1):
    wid = lax.axis_index("s") * 2 + lax.axis_index("c")
    nrows = x_hbm.shape[0] // _NW
    row0 = wid * nrows
    nch = nrows // _CHROWS
    T = x_hbm.shape[1]

    def chunk_copy(r, buf, sem):
        copies = [
            pltpu.make_async_copy(
                x_hbm.at[r + q], buf.at[pl.ds(q * T, T)], sem)
            for q in range(_CHROWS)
        ]
        return copies

    def start(copies):
        for c in copies:
            c.start()

    def drain(copies):
        for c in copies:
            c.wait()

    start(chunk_copy(row0, buf0, sem0))

    def step(i, acc):
        r = row0 + 2 * i * _CHROWS
        start(chunk_copy(r + _CHROWS, buf1, sem1))
        drain(chunk_copy(r, buf0, sem0))
        acc = _sc_process_chunk(buf0, acc)

        @pl.when(2 * i + 2 < nch)
        def _():
            start(chunk_copy(r + 2 * _CHROWS, buf0, sem0))

        drain(chunk_copy(r + _CHROWS, buf1, sem1))
        acc = _sc_process_chunk(buf1, acc)
        return acc

    acc = lax.fori_loop(0, nch // 2, step,
                        jnp.zeros((_LANES,), jnp.float32))
    accv[...] = acc
    pltpu.sync_copy(accv, out_hbm.at[wid])


def _sc_top2_partials(x):
    mesh = plsc.VectorSubcoreMesh(core_axis_name="c", subcore_axis_name="s")
    T = x.shape[1]
    run = pl.kernel(
        _sc_kernel_body,
        out_type=jax.ShapeDtypeStruct((_NW, _LANES), jnp.float32),
        mesh=mesh,
        scratch_types=[
            pltpu.VMEM((_CHROWS * T,), jnp.float32),
            pltpu.VMEM((_CHROWS * T,), jnp.float32),
            pltpu.VMEM((_LANES,), jnp.float32),
            pltpu.SemaphoreType.DMA,
            pltpu.SemaphoreType.DMA,
        ],
        compiler_params=pltpu.CompilerParams(needs_layout_passes=False),
    )
    return run(x)


def kernel(pred_logits):
    parts = _sc_top2_partials(pred_logits)
    sp_sum = _tc_softplus_sum(pred_logits)
    return sp_sum - jnp.sum(parts)
